# Initial kernel scaffold; baseline (speedup 1.0000x reference)
import jax
import jax.numpy as jnp
from jax.experimental import pallas as pl


def kernel(indices, table):
    raise NotImplementedError("write your pallas kernel here")


# speedup vs baseline: 1.0233x; 1.0233x over previous
import functools

import jax
import jax.numpy as jnp
from jax import lax
from jax.experimental import pallas as pl
from jax.experimental.pallas import tpu as pltpu
from jax.experimental.pallas import tpu_sc as plsc

_CH = 128


@functools.lru_cache(maxsize=None)
def _make_gather(total, n_dim, nc, ns, n_chunks):
    mesh = plsc.VectorSubcoreMesh(core_axis_name="c", subcore_axis_name="s")

    @functools.partial(
        pl.kernel,
        out_type=jax.ShapeDtypeStruct((total, n_dim), jnp.float32),
        mesh=mesh,
        scratch_types=[
            pltpu.VMEM((n_chunks, _CH), jnp.int32),
            pltpu.VMEM((_CH, n_dim), jnp.float32),
            pltpu.SemaphoreType.DMA,
        ],
        compiler_params=pltpu.CompilerParams(use_tc_tiling_on_sc=False),
    )
    def k(idx_hbm, table_hbm, out_hbm, idx_v, rows_v, sem):
        wid = lax.axis_index("s") * nc + lax.axis_index("c")
        base = wid * (n_chunks * _CH)
        pltpu.sync_copy(idx_hbm.at[wid], idx_v)

        def body(j, carry):
            pltpu.async_copy(table_hbm.at[idx_v.at[j]], rows_v, sem).wait()
            pltpu.sync_copy(rows_v, out_hbm.at[pl.ds(base + j * _CH, _CH)])
            return carry

        lax.fori_loop(0, n_chunks, body, 0)

    return k


def kernel(indices, table):
    b, h = indices.shape
    n_classes, n_dim = table.shape
    total = b * h
    info = plsc.get_sparse_core_info()
    nc, ns = info.num_cores, info.num_subcores
    nw = nc * ns
    n_chunks = total // (nw * _CH)
    idx = indices.reshape(nw, n_chunks, _CH).astype(jnp.int32)
    out = _make_gather(total, n_dim, nc, ns, n_chunks)(idx, table)
    return out.reshape(b, h, n_dim)


# speedup vs baseline: 1.1127x; 1.0874x over previous
import functools

import jax
import jax.numpy as jnp
from jax import lax
from jax.experimental import pallas as pl
from jax.experimental.pallas import tpu as pltpu
from jax.experimental.pallas import tpu_sc as plsc

_CH = 128
_NBUF = 8


@functools.lru_cache(maxsize=None)
def _make_gather(total, n_dim, nc, ns, n_chunks):
    mesh = plsc.VectorSubcoreMesh(core_axis_name="c", subcore_axis_name="s")
    n_groups = n_chunks // _NBUF
    assert n_chunks % _NBUF == 0 and n_groups >= 2

    scratch = (
        [pltpu.VMEM((n_chunks, _CH), jnp.int32)]
        + [pltpu.VMEM((_CH, n_dim), jnp.float32) for _ in range(_NBUF)]
        + [pltpu.SemaphoreType.DMA for _ in range(2 * _NBUF)]
    )

    @functools.partial(
        pl.kernel,
        out_type=jax.ShapeDtypeStruct((total, n_dim), jnp.float32),
        mesh=mesh,
        scratch_types=scratch,
        compiler_params=pltpu.CompilerParams(use_tc_tiling_on_sc=False),
    )
    def k(idx_hbm, table_hbm, out_hbm, idx_v, *bufs_and_sems):
        rows = bufs_and_sems[:_NBUF]
        gsem = bufs_and_sems[_NBUF:2 * _NBUF]
        ssem = bufs_and_sems[2 * _NBUF:]
        wid = lax.axis_index("s") * nc + lax.axis_index("c")
        base = wid * (n_chunks * _CH)
        pltpu.sync_copy(idx_hbm.at[wid], idx_v)

        def fire_gather(j, b):
            pltpu.async_copy(table_hbm.at[idx_v.at[j]], rows[b], gsem[b])

        def wait_gather(b):
            pltpu.make_async_copy(
                table_hbm.at[idx_v.at[0]], rows[b], gsem[b]).wait()

        def fire_store(j, b):
            pltpu.async_copy(
                rows[b], out_hbm.at[pl.ds(base + j * _CH, _CH)], ssem[b])

        def wait_store(b):
            pltpu.make_async_copy(
                rows[b], out_hbm.at[pl.ds(base, _CH)], ssem[b]).wait()

        for b in range(_NBUF):
            fire_gather(b, b)

        def body(gi, carry):
            g = gi * _NBUF
            for b in range(_NBUF):
                wait_gather(b)
                fire_store(g + b, b)
            for b in range(_NBUF):
                wait_store(b)
                fire_gather(g + _NBUF + b, b)
            return carry

        lax.fori_loop(0, n_groups - 1, body, 0)

        g = (n_groups - 1) * _NBUF
        for b in range(_NBUF):
            wait_gather(b)
            fire_store(g + b, b)
        for b in range(_NBUF):
            wait_store(b)

    return k


def kernel(indices, table):
    b, h = indices.shape
    n_classes, n_dim = table.shape
    total = b * h
    info = plsc.get_sparse_core_info()
    nc, ns = info.num_cores, info.num_subcores
    nw = nc * ns
    n_chunks = total // (nw * _CH)
    idx = indices.reshape(nw, n_chunks, _CH).astype(jnp.int32)
    out = _make_gather(total, n_dim, nc, ns, n_chunks)(idx, table)
    return out.reshape(b, h, n_dim)


# speedup vs baseline: 1.7719x; 1.5924x over previous
import functools

import jax
import jax.numpy as jnp
from jax import lax
from jax.experimental import pallas as pl
from jax.experimental.pallas import tpu as pltpu
from jax.experimental.pallas import tpu_sc as plsc

_NBUF = 8
_GR = 1


@functools.lru_cache(maxsize=None)
def _make_gather(batch, hist, n_dim, nc, ns):
    mesh = plsc.VectorSubcoreMesh(core_axis_name="c", subcore_axis_name="s")
    nw = nc * ns
    rows_w = batch // nw
    n_chunks = rows_w // _GR
    n_groups = n_chunks // _NBUF
    assert rows_w % _GR == 0 and n_chunks % _NBUF == 0 and n_groups >= 2

    scratch = (
        [pltpu.VMEM((rows_w, hist), jnp.int32)]
        + [pltpu.VMEM((hist, n_dim), jnp.float32) for _ in range(_NBUF)]
        + [pltpu.SemaphoreType.DMA for _ in range(2 * _NBUF)]
    )

    @functools.partial(
        pl.kernel,
        out_type=jax.ShapeDtypeStruct((batch, hist, n_dim), jnp.float32),
        mesh=mesh,
        scratch_types=scratch,
        compiler_params=pltpu.CompilerParams(use_tc_tiling_on_sc=False),
    )
    def k(idx_hbm, table_hbm, out_hbm, idx_v, *bufs_and_sems):
        rows = bufs_and_sems[:_NBUF]
        gsem = bufs_and_sems[_NBUF:2 * _NBUF]
        ssem = bufs_and_sems[2 * _NBUF:]
        wid = lax.axis_index("s") * nc + lax.axis_index("c")
        base = wid * rows_w
        pltpu.sync_copy(idx_hbm.at[pl.ds(base, rows_w)], idx_v)

        def fire_gather(j, b):
            pltpu.async_copy(table_hbm.at[idx_v.at[j]], rows[b], gsem[b])

        def wait_gather(b):
            pltpu.make_async_copy(
                table_hbm.at[idx_v.at[0]], rows[b], gsem[b]).wait()

        def fire_store(j, b):
            pltpu.async_copy(rows[b], out_hbm.at[base + j], ssem[b])

        def wait_store(b):
            pltpu.make_async_copy(rows[b], out_hbm.at[base], ssem[b]).wait()

        for b in range(_NBUF):
            fire_gather(b, b)

        def body(gi, carry):
            g = gi * _NBUF
            for b in range(_NBUF):
                wait_gather(b)
                fire_store(g + b, b)
            for b in range(_NBUF):
                wait_store(b)
                fire_gather(g + _NBUF + b, b)
            return carry

        lax.fori_loop(0, n_groups - 1, body, 0)

        g = (n_groups - 1) * _NBUF
        for b in range(_NBUF):
            wait_gather(b)
            fire_store(g + b, b)
        for b in range(_NBUF):
            wait_store(b)

    return k


def kernel(indices, table):
    batch, hist = indices.shape
    n_classes, n_dim = table.shape
    info = plsc.get_sparse_core_info()
    nc, ns = info.num_cores, info.num_subcores
    return _make_gather(batch, hist, n_dim, nc, ns)(indices, table)


# speedup vs baseline: 2.4747x; 1.3966x over previous
import functools

import jax
import jax.numpy as jnp
from jax import lax
from jax.experimental import pallas as pl
from jax.experimental.pallas import tpu as pltpu
from jax.experimental.pallas import tpu_sc as plsc

_NBUF = 8
_GR = 1


@functools.lru_cache(maxsize=None)
def _make_gather(batch, hist, n_dim, nc, ns):
    mesh = plsc.VectorSubcoreMesh(core_axis_name="c", subcore_axis_name="s")
    nw = nc * ns
    rows_w = batch // nw
    n_chunks = rows_w // _GR
    n_groups = n_chunks // _NBUF
    assert rows_w % _GR == 0 and n_chunks % _NBUF == 0 and n_groups >= 2

    scratch = (
        [pltpu.VMEM((rows_w, hist), jnp.int32)]
        + [pltpu.VMEM((hist, n_dim), jnp.float32) for _ in range(_NBUF)]
        + [pltpu.SemaphoreType.DMA for _ in range(2 * _NBUF)]
    )

    hp = (hist + 7) // 8 * 8
    dp = 128

    @functools.partial(
        pl.kernel,
        out_type=jax.ShapeDtypeStruct((batch, hp, dp), jnp.float32),
        mesh=mesh,
        scratch_types=scratch,
        compiler_params=pltpu.CompilerParams(use_tc_tiling_on_sc=False),
    )
    def k(idx_hbm, table_hbm, out_hbm, idx_v, *bufs_and_sems):
        rows = bufs_and_sems[:_NBUF]
        gsem = bufs_and_sems[_NBUF:2 * _NBUF]
        ssem = bufs_and_sems[2 * _NBUF:]
        wid = lax.axis_index("s") * nc + lax.axis_index("c")
        base = wid * rows_w
        pltpu.sync_copy(idx_hbm.at[pl.ds(base, rows_w)], idx_v)

        def fire_gather(j, b):
            pltpu.async_copy(table_hbm.at[idx_v.at[j]], rows[b], gsem[b])

        def wait_gather(b):
            pltpu.make_async_copy(
                table_hbm.at[idx_v.at[0]], rows[b], gsem[b]).wait()

        def fire_store(j, b):
            pltpu.async_copy(
                rows[b],
                out_hbm.at[base + j, pl.ds(0, hist), pl.ds(0, n_dim)],
                ssem[b])

        def wait_store(b):
            pltpu.make_async_copy(
                rows[b],
                out_hbm.at[base, pl.ds(0, hist), pl.ds(0, n_dim)],
                ssem[b]).wait()

        for b in range(_NBUF):
            fire_gather(b, b)

        def body(gi, carry):
            g = gi * _NBUF
            for b in range(_NBUF):
                wait_gather(b)
                fire_store(g + b, b)
            for b in range(_NBUF):
                wait_store(b)
                fire_gather(g + _NBUF + b, b)
            return carry

        lax.fori_loop(0, n_groups - 1, body, 0)

        g = (n_groups - 1) * _NBUF
        for b in range(_NBUF):
            wait_gather(b)
            fire_store(g + b, b)
        for b in range(_NBUF):
            wait_store(b)

    return k


def kernel(indices, table):
    batch, hist = indices.shape
    n_classes, n_dim = table.shape
    info = plsc.get_sparse_core_info()
    nc, ns = info.num_cores, info.num_subcores
    padded = _make_gather(batch, hist, n_dim, nc, ns)(indices, table)
    return padded[:, :hist, :n_dim]
